# baseline (speedup 1.0000x reference)
import numpy as np
import jax
import jax.numpy as jnp
from jax import lax
from jax.experimental import pallas as pl

_ROT = ((13, 15, 26, 6), (17, 29, 16, 24))


def _np_threefry(key, x0, x1):
    ks0, ks1 = np.uint32(key[0]), np.uint32(key[1])
    ks2 = np.uint32(ks0 ^ ks1 ^ np.uint32(0x1BD11BDA))
    ks = (ks0, ks1, ks2)
    x0 = (x0 + ks0).astype(np.uint32)
    x1 = (x1 + ks1).astype(np.uint32)
    for i in range(5):
        for r in _ROT[i % 2]:
            x0 = (x0 + x1).astype(np.uint32)
            x1 = ((x1 << np.uint32(r)) | (x1 >> np.uint32(32 - r))).astype(np.uint32)
            x1 = (x1 ^ x0).astype(np.uint32)
        x0 = (x0 + ks[(i + 1) % 3]).astype(np.uint32)
        x1 = (x1 + ks[(i + 2) % 3] + np.uint32(i + 1)).astype(np.uint32)
    return x0, x1


def _np_split(key, num):
    o0, o1 = _np_threefry(key, np.zeros(num, np.uint32), np.arange(num, dtype=np.uint32))
    return [(int(o0[i]), int(o1[i])) for i in range(num)]


_K1, _K2, _K3, _K4, _K5 = _np_split((0, 42), 5)
_K1A, _K1B = _np_split(_K1, 2)
_K4A, _K4B = _np_split(_K4, 2)
_K5A, _K5B = _np_split(_K5, 2)

_MAX_OBJ = 100


def _bits(key, shape, counter=None):
    if counter is None:
        r = lax.broadcasted_iota(jnp.uint32, shape, 0)
        c = lax.broadcasted_iota(jnp.uint32, shape, 1)
        x1 = r * jnp.uint32(shape[1]) + c
    else:
        x1 = counter
    ks0 = int(key[0])
    ks1 = int(key[1])
    ks2 = ks0 ^ ks1 ^ 0x1BD11BDA
    ks = (ks0, ks1, ks2)
    x0 = jnp.full(shape, jnp.uint32(ks0), jnp.uint32)
    x1 = x1 + jnp.uint32(ks1)
    for i in range(5):
        for rot in _ROT[i % 2]:
            x0 = x0 + x1
            x1 = (x1 << jnp.uint32(rot)) | (x1 >> jnp.uint32(32 - rot))
            x1 = x1 ^ x0
        x0 = x0 + jnp.uint32(ks[(i + 1) % 3])
        x1 = x1 + jnp.uint32((ks[(i + 2) % 3] + i + 1) & 0xFFFFFFFF)
    return x0 ^ x1


def _mod_small(x, span):
    q = jnp.floor(x.astype(jnp.float32) * np.float32(1.0 / span)).astype(jnp.int32)
    r = x - q * jnp.int32(span)
    r = jnp.where(r < 0, r + jnp.int32(span), r)
    r = jnp.where(r < 0, r + jnp.int32(span), r)
    r = jnp.where(r >= jnp.int32(span), r - jnp.int32(span), r)
    r = jnp.where(r >= jnp.int32(span), r - jnp.int32(span), r)
    return r


def _fold16(x, c16):
    return (x >> 16) * c16 + (x & 0xFFFF)


def _mod_u32(u, span):
    c16 = (1 << 16) % span
    x = _fold16(u, jnp.uint32(c16))
    bound = (1 << 16) * c16 + (1 << 16)
    x = x.astype(jnp.int32)
    cf = jnp.int32(c16)
    while bound >= (1 << 24):
        x = _fold16(x, cf)
        bound = (bound >> 16) * c16 + (1 << 16)
    return _mod_small(x, span)


def _randint_bits(key_a, key_b, shape, span, counter=None):
    u = _bits(key_a, shape, counter)
    v = _bits(key_b, shape, counter)
    mult = ((1 << 16) % span) ** 2 % span
    t = _mod_u32(u, span) * jnp.int32(mult) + _mod_u32(v, span)
    bound = span * mult + span
    c16 = (1 << 16) % span
    while bound >= (1 << 24):
        t = _fold16(t, jnp.int32(c16))
        bound = (bound >> 16) * c16 + (1 << 16)
    return _mod_small(t, span)


def _erfinv(x):
    w = -jnp.log1p(-x * x)
    ws = w - np.float32(2.5)
    p = jnp.full_like(x, np.float32(2.81022636e-08))
    for c in (3.43273939e-07, -3.5233877e-06, -4.39150654e-06, 0.00021858087,
              -0.00125372503, -0.00417768164, 0.246640727, 1.50140941):
        p = np.float32(c) + p * ws
    wl = jnp.sqrt(w) - np.float32(3.0)
    q = jnp.full_like(x, np.float32(-0.000200214257))
    for c in (0.000100950558, 0.00134934322, -0.00367342844, 0.00573950773,
              -0.0076224613, 0.00943887047, 1.00167406, 2.83297682):
        q = np.float32(c) + q * wl
    return jnp.where(w < np.float32(5.0), p, q) * x


def _normal(key, shape, counter=None):
    bits = _bits(key, shape, counter)
    f = lax.bitcast_convert_type((bits >> 9) | jnp.uint32(0x3F800000), jnp.float32)
    f = f - np.float32(1.0)
    lo = np.float32(-0.9999999403953552)
    hi = np.float32(1.0)
    u = jnp.maximum(lo, f * (hi - lo) + lo)
    return np.float32(np.sqrt(np.float32(2.0))) * _erfinv(u)


def kernel(x):
    batch, chans, num_boxes = x.shape
    num_classes = chans - 4

    bx_rows = batch * 4 // 2
    half = batch // 2

    def row2(shape, pid, rows_per_step):
        r = lax.broadcasted_iota(jnp.uint32, shape, 0) + jnp.uint32(rows_per_step) * pid
        c = lax.broadcasted_iota(jnp.uint32, shape, 1)
        return r, c

    def gen_body(nd_ref, bx_ref, sc_ref, cl_ref, ix_ref):
        pid = pl.program_id(0).astype(jnp.uint32)

        @pl.when(pl.program_id(0) == 0)
        def _():
            nd_ref[...] = _randint_bits(_K1A, _K1B, (1, batch), _MAX_OBJ).reshape(batch)

        r, j = row2(bx_ref.shape, pid, bx_rows)
        cnt = (r >> 2) * jnp.uint32(4 * _MAX_OBJ) + j * 4 + (r & 3)
        bx_ref[...] = _normal(_K2, bx_ref.shape, cnt)

        r, c = row2(sc_ref.shape, pid, half)
        sc_ref[...] = _normal(_K3, sc_ref.shape, r * jnp.uint32(_MAX_OBJ) + c)
        r, c = row2(cl_ref.shape, pid, half)
        cl_ref[...] = _randint_bits(_K4A, _K4B, cl_ref.shape, num_classes,
                                    r * jnp.uint32(_MAX_OBJ) + c)
        r, c = row2(ix_ref.shape, pid, half)
        ix_ref[...] = _randint_bits(_K5A, _K5B, ix_ref.shape, num_boxes,
                                    r * jnp.uint32(_MAX_OBJ) + c)

    nd, bx, sc, cl, ix = pl.pallas_call(
        gen_body,
        grid=(2,),
        out_specs=[
            pl.BlockSpec((batch,), lambda i: (0,)),
            pl.BlockSpec((bx_rows, _MAX_OBJ), lambda i: (i, 0)),
            pl.BlockSpec((half, _MAX_OBJ), lambda i: (i, 0)),
            pl.BlockSpec((half, _MAX_OBJ), lambda i: (i, 0)),
            pl.BlockSpec((half, _MAX_OBJ), lambda i: (i, 0)),
        ],
        out_shape=[
            jax.ShapeDtypeStruct((batch,), jnp.int32),
            jax.ShapeDtypeStruct((batch * 4, _MAX_OBJ), jnp.float32),
            jax.ShapeDtypeStruct((batch, _MAX_OBJ), jnp.float32),
            jax.ShapeDtypeStruct((batch, _MAX_OBJ), jnp.int32),
            jax.ShapeDtypeStruct((batch, _MAX_OBJ), jnp.int32),
        ],
    )()
    boxes = bx.reshape(batch, 4, _MAX_OBJ).transpose(0, 2, 1)
    return nd.reshape(batch, 1), boxes, sc, cl, ix


# speedup vs baseline: 1.0424x; 1.0424x over previous
import numpy as np
import jax
import jax.numpy as jnp
from jax import lax
from jax.experimental import pallas as pl

_ROT = ((13, 15, 26, 6), (17, 29, 16, 24))


def _np_threefry(key, x0, x1):
    ks0, ks1 = np.uint32(key[0]), np.uint32(key[1])
    ks2 = np.uint32(ks0 ^ ks1 ^ np.uint32(0x1BD11BDA))
    ks = (ks0, ks1, ks2)
    x0 = (x0 + ks0).astype(np.uint32)
    x1 = (x1 + ks1).astype(np.uint32)
    for i in range(5):
        for r in _ROT[i % 2]:
            x0 = (x0 + x1).astype(np.uint32)
            x1 = ((x1 << np.uint32(r)) | (x1 >> np.uint32(32 - r))).astype(np.uint32)
            x1 = (x1 ^ x0).astype(np.uint32)
        x0 = (x0 + ks[(i + 1) % 3]).astype(np.uint32)
        x1 = (x1 + ks[(i + 2) % 3] + np.uint32(i + 1)).astype(np.uint32)
    return x0, x1


def _np_split(key, num):
    o0, o1 = _np_threefry(key, np.zeros(num, np.uint32), np.arange(num, dtype=np.uint32))
    return [(int(o0[i]), int(o1[i])) for i in range(num)]


_K1, _K2, _K3, _K4, _K5 = _np_split((0, 42), 5)
_K1A, _K1B = _np_split(_K1, 2)
_K4A, _K4B = _np_split(_K4, 2)
_K5A, _K5B = _np_split(_K5, 2)

_MAX_OBJ = 100


def _bits(key, shape, counter=None):
    if counter is None:
        r = lax.broadcasted_iota(jnp.uint32, shape, 0)
        c = lax.broadcasted_iota(jnp.uint32, shape, 1)
        x1 = r * jnp.uint32(shape[1]) + c
    else:
        x1 = counter
    ks0 = int(key[0])
    ks1 = int(key[1])
    ks2 = ks0 ^ ks1 ^ 0x1BD11BDA
    ks = (ks0, ks1, ks2)
    x0 = jnp.full(shape, jnp.uint32(ks0), jnp.uint32)
    x1 = x1 + jnp.uint32(ks1)
    for i in range(5):
        for rot in _ROT[i % 2]:
            x0 = x0 + x1
            x1 = (x1 << jnp.uint32(rot)) | (x1 >> jnp.uint32(32 - rot))
            x1 = x1 ^ x0
        x0 = x0 + jnp.uint32(ks[(i + 1) % 3])
        x1 = x1 + jnp.uint32((ks[(i + 2) % 3] + i + 1) & 0xFFFFFFFF)
    return x0 ^ x1


def _mod_small(x, span):
    q = jnp.floor(x.astype(jnp.float32) * np.float32(1.0 / span)).astype(jnp.int32)
    r = x - q * jnp.int32(span)
    r = jnp.where(r < 0, r + jnp.int32(span), r)
    r = jnp.where(r < 0, r + jnp.int32(span), r)
    r = jnp.where(r >= jnp.int32(span), r - jnp.int32(span), r)
    r = jnp.where(r >= jnp.int32(span), r - jnp.int32(span), r)
    return r


def _fold16(x, c16):
    return (x >> 16) * c16 + (x & 0xFFFF)


def _mod_u32(u, span):
    c16 = (1 << 16) % span
    x = _fold16(u, jnp.uint32(c16))
    bound = (1 << 16) * c16 + (1 << 16)
    x = x.astype(jnp.int32)
    cf = jnp.int32(c16)
    while bound >= (1 << 24):
        x = _fold16(x, cf)
        bound = (bound >> 16) * c16 + (1 << 16)
    return _mod_small(x, span)


def _randint_bits(key_a, key_b, shape, span):
    u = _bits(key_a, shape)
    v = _bits(key_b, shape)
    mult = ((1 << 16) % span) ** 2 % span
    t = _mod_u32(u, span) * jnp.int32(mult) + _mod_u32(v, span)
    bound = span * mult + span
    c16 = (1 << 16) % span
    while bound >= (1 << 24):
        t = _fold16(t, jnp.int32(c16))
        bound = (bound >> 16) * c16 + (1 << 16)
    return _mod_small(t, span)


def _erfinv(x):
    w = -jnp.log1p(-x * x)
    ws = w - np.float32(2.5)
    p = jnp.full_like(x, np.float32(2.81022636e-08))
    for c in (3.43273939e-07, -3.5233877e-06, -4.39150654e-06, 0.00021858087,
              -0.00125372503, -0.00417768164, 0.246640727, 1.50140941):
        p = np.float32(c) + p * ws
    wl = jnp.sqrt(w) - np.float32(3.0)
    q = jnp.full_like(x, np.float32(-0.000200214257))
    for c in (0.000100950558, 0.00134934322, -0.00367342844, 0.00573950773,
              -0.0076224613, 0.00943887047, 1.00167406, 2.83297682):
        q = np.float32(c) + q * wl
    return jnp.where(w < np.float32(5.0), p, q) * x


def _normal(key, shape, counter=None):
    bits = _bits(key, shape, counter)
    f = lax.bitcast_convert_type((bits >> 9) | jnp.uint32(0x3F800000), jnp.float32)
    f = f - np.float32(1.0)
    lo = np.float32(-0.9999999403953552)
    hi = np.float32(1.0)
    u = jnp.maximum(lo, f * (hi - lo) + lo)
    return np.float32(np.sqrt(np.float32(2.0))) * _erfinv(u)


def kernel(x):
    batch, chans, num_boxes = x.shape
    num_classes = chans - 4

    def gen_body(nd_ref, bx_ref, sc_ref, cl_ref, ix_ref):
        nd_ref[...] = _randint_bits(_K1A, _K1B, (1, batch), _MAX_OBJ).reshape(batch)
        r = lax.broadcasted_iota(jnp.uint32, bx_ref.shape, 0)
        j = lax.broadcasted_iota(jnp.uint32, bx_ref.shape, 1)
        cnt = (r >> 2) * jnp.uint32(4 * _MAX_OBJ) + j * 4 + (r & 3)
        bx_ref[...] = _normal(_K2, bx_ref.shape, cnt)
        sc_ref[...] = _normal(_K3, sc_ref.shape)
        cl_ref[...] = _randint_bits(_K4A, _K4B, cl_ref.shape, num_classes)
        ix_ref[...] = _randint_bits(_K5A, _K5B, ix_ref.shape, num_boxes)

    nd, bx, sc, cl, ix = pl.pallas_call(
        gen_body,
        out_shape=[
            jax.ShapeDtypeStruct((batch,), jnp.int32),
            jax.ShapeDtypeStruct((batch * 4, _MAX_OBJ), jnp.float32),
            jax.ShapeDtypeStruct((batch, _MAX_OBJ), jnp.float32),
            jax.ShapeDtypeStruct((batch, _MAX_OBJ), jnp.int32),
            jax.ShapeDtypeStruct((batch, _MAX_OBJ), jnp.int32),
        ],
    )()
    boxes = bx.reshape(batch, 4, _MAX_OBJ).transpose(0, 2, 1)
    return nd.reshape(batch, 1), boxes, sc, cl, ix
